# Initial kernel scaffold; baseline (speedup 1.0000x reference)
import jax
import jax.numpy as jnp
from jax.experimental import pallas as pl


def kernel(x, edge_index, W1, b1, W2, b2, W3, b3, Wc1, bc1, Wc2, bc2):
    raise NotImplementedError("write your pallas kernel here")


# speedup vs baseline: 8.8450x; 8.8450x over previous
import functools

import jax
import jax.numpy as jnp
from jax import lax
from jax.experimental import pallas as pl
from jax.experimental.pallas import tpu as pltpu
from jax.experimental.pallas import tpu_sc as plsc

NC, NS, L = 2, 16, 16
NW = NC * NS
K = 128


def _sc_mesh():
    return plsc.VectorSubcoreMesh(
        core_axis_name="c", subcore_axis_name="s", num_cores=NC, num_subcores=NS
    )


def _edge_scatter(hs, src_p, dst_p, zacc, n_acc, feat, chunks_per_tile):
    rows_pt = n_acc // NS

    @functools.partial(
        pl.kernel,
        out_type=jax.ShapeDtypeStruct((NC, n_acc, feat), jnp.float32),
        mesh=_sc_mesh(),
        scratch_types=[
            pltpu.VMEM((K,), jnp.int32),
            pltpu.VMEM((K,), jnp.int32),
            pltpu.VMEM((K, feat), jnp.float32),
            pltpu.VMEM_SHARED((n_acc, feat), jnp.float32),
            pltpu.SemaphoreType.DMA,
        ],
    )
    def k(hs_hbm, src_hbm, dst_hbm, z_hbm, out_hbm, src_v, dst_v, rows_v, acc, sem):
        cid = lax.axis_index("c")
        sid = lax.axis_index("s")
        wid = sid * NC + cid
        r0 = sid * rows_pt
        pltpu.sync_copy(z_hbm.at[pl.ds(r0, rows_pt)], acc.at[pl.ds(r0, rows_pt)])
        plsc.subcore_barrier()
        base0 = wid * chunks_per_tile * K

        def body(j, c):
            base = base0 + j * K
            pltpu.sync_copy(src_hbm.at[pl.ds(base, K)], src_v)
            pltpu.sync_copy(dst_hbm.at[pl.ds(base, K)], dst_v)
            pltpu.async_copy(hs_hbm.at[src_v], rows_v, sem).wait()
            pltpu.sync_copy(rows_v, acc.at[dst_v], add=True)
            return c

        lax.fori_loop(0, chunks_per_tile, body, 0)
        plsc.subcore_barrier()
        pltpu.sync_copy(acc.at[pl.ds(r0, rows_pt)], out_hbm.at[cid, pl.ds(r0, rows_pt)])

    return k(hs, src_p, dst_p, zacc)


def _deg_count(dst_p, ones_rows, zacc, n_acc, feat, chunks_per_tile):
    rows_pt = n_acc // NS

    @functools.partial(
        pl.kernel,
        out_type=jax.ShapeDtypeStruct((NC, n_acc, feat), jnp.float32),
        mesh=_sc_mesh(),
        scratch_types=[
            pltpu.VMEM((K,), jnp.int32),
            pltpu.VMEM((K, feat), jnp.float32),
            pltpu.VMEM_SHARED((n_acc, feat), jnp.float32),
        ],
    )
    def k(dst_hbm, ones_hbm, z_hbm, out_hbm, dst_v, ones_v, acc):
        cid = lax.axis_index("c")
        sid = lax.axis_index("s")
        wid = sid * NC + cid
        r0 = sid * rows_pt
        pltpu.sync_copy(ones_hbm, ones_v)
        pltpu.sync_copy(z_hbm.at[pl.ds(r0, rows_pt)], acc.at[pl.ds(r0, rows_pt)])
        plsc.subcore_barrier()
        base0 = wid * chunks_per_tile * K

        def body(j, c):
            base = base0 + j * K
            pltpu.sync_copy(dst_hbm.at[pl.ds(base, K)], dst_v)
            pltpu.sync_copy(ones_v, acc.at[dst_v], add=True)
            return c

        lax.fori_loop(0, chunks_per_tile, body, 0)
        plsc.subcore_barrier()
        pltpu.sync_copy(acc.at[pl.ds(r0, rows_pt)], out_hbm.at[cid, pl.ds(r0, rows_pt)])

    return k(dst_p, ones_rows, zacc)


def _tc_first(d0, d1, x, W):
    n, h = x.shape[0], W.shape[1]

    def body(d0r, d1r, xr, wr, hsr, disr):
        dis = lax.rsqrt(d0r[...] + d1r[...] + 1.0)
        mm = jnp.dot(xr[...], wr[...], preferred_element_type=jnp.float32)
        hsr[...] = mm * dis
        disr[...] = dis

    return pl.pallas_call(
        body,
        out_shape=[
            jax.ShapeDtypeStruct((n, h), jnp.float32),
            jax.ShapeDtypeStruct((n, h), jnp.float32),
        ],
    )(d0, d1, x, W)


def _tc_mid(p0, p1, hs_prev, dis, b2d, W):
    n, h = hs_prev.shape[0], W.shape[1]

    def body(p0r, p1r, hpr, disr, br, wr, outr):
        dis = disr[...]
        hh = jnp.maximum(dis * (p0r[...] + p1r[...] + hpr[...]) + br[...], 0.0)
        outr[...] = jnp.dot(hh, wr[...], preferred_element_type=jnp.float32) * dis

    return pl.pallas_call(
        body, out_shape=jax.ShapeDtypeStruct((n, h), jnp.float32)
    )(p0, p1, hs_prev, dis, b2d, W)


def _tc_final(p0, p1, hs3, dis, b3, Wc1, bc1, Wc2, bc2):
    n, c = hs3.shape[0], Wc2.shape[1]

    def body(p0r, p1r, hpr, disr, br, w1r, b1r, w2r, b2r, outr):
        hh = jnp.maximum(disr[...] * (p0r[...] + p1r[...] + hpr[...]) + br[...], 0.0)
        m = jnp.maximum(
            jnp.dot(hh, w1r[...], preferred_element_type=jnp.float32) + b1r[...], 0.0
        )
        logits = jnp.dot(m, w2r[...], preferred_element_type=jnp.float32) + b2r[...]
        mx = jnp.max(logits, axis=1, keepdims=True)
        lse = mx + jnp.log(jnp.sum(jnp.exp(logits - mx), axis=1, keepdims=True))
        outr[...] = logits - lse

    return pl.pallas_call(
        body, out_shape=jax.ShapeDtypeStruct((n, c), jnp.float32)
    )(p0, p1, hs3, dis, b3, Wc1, bc1, Wc2, bc2)


def kernel(x, edge_index, W1, b1, W2, b2, W3, b3, Wc1, bc1, Wc2, bc2):
    N, _ = x.shape
    H = W1.shape[1]
    E = edge_index.shape[1]
    FW = 128
    pad_h = FW - H
    W1p = jnp.pad(W1, ((0, 0), (0, pad_h)))
    W2p = jnp.pad(W2, ((0, pad_h), (0, pad_h)))
    W3p = jnp.pad(W3, ((0, pad_h), (0, pad_h)))
    Wc1p = jnp.pad(Wc1, ((0, pad_h), (0, 0)))
    b1p = jnp.pad(b1, (0, pad_h)).reshape(1, FW)
    b2p = jnp.pad(b2, (0, pad_h)).reshape(1, FW)
    b3p = jnp.pad(b3, (0, pad_h)).reshape(1, FW)
    n_acc = -(-(N + 1) // (NS * 8)) * (NS * 8)

    src = edge_index[0]
    dst = edge_index[1]
    cpt = -(-E // (NW * K))
    e_pad = cpt * NW * K
    pad = e_pad - E
    src_p = jnp.concatenate([src, jnp.zeros((pad,), jnp.int32)])
    dst_p = jnp.concatenate([dst, jnp.full((pad,), N, jnp.int32)])

    z_feat = jnp.zeros((n_acc, FW), jnp.float32)
    ones_rows = jnp.ones((K, FW), jnp.float32)

    deg_parts = _deg_count(dst_p, ones_rows, z_feat, n_acc, FW, cpt)
    d0 = deg_parts[0, :N]
    d1 = deg_parts[1, :N]

    hs1, dis = _tc_first(d0, d1, x, W1p)

    P = _edge_scatter(hs1, src_p, dst_p, z_feat, n_acc, FW, cpt)
    hs2 = _tc_mid(P[0, :N], P[1, :N], hs1, dis, b1p, W2p)

    P = _edge_scatter(hs2, src_p, dst_p, z_feat, n_acc, FW, cpt)
    hs3 = _tc_mid(P[0, :N], P[1, :N], hs2, dis, b2p, W3p)

    P = _edge_scatter(hs3, src_p, dst_p, z_feat, n_acc, FW, cpt)
    return _tc_final(
        P[0, :N], P[1, :N], hs3, dis, b3p,
        Wc1p, bc1.reshape(1, -1), Wc2, bc2.reshape(1, -1),
    )
